# baseline (speedup 1.0000x reference)
import math

import jax
import jax.numpy as jnp
from jax import lax
from jax.experimental import pallas as pl
from jax.experimental.pallas import tpu as pltpu
from jax.experimental.pallas import tpu_sc as plsc

T = 32768
D_FEAT = 32
K = 16
NW = 32
CHUNK = T // NW


def _sc_body(x_hbm, mu_t_hbm, ls_t_hbm, out_hbm, x_v, out_v, mu_v, ls_v):
    wid = lax.axis_index("s") * 2 + lax.axis_index("c")
    base = wid * CHUNK

    pltpu.sync_copy(x_hbm.at[pl.ds(base * D_FEAT, CHUNK * D_FEAT)], x_v)
    pltpu.sync_copy(mu_t_hbm, mu_v)
    pltpu.sync_copy(ls_t_hbm, ls_v)

    c = jnp.full((K,), -0.5 * D_FEAT * math.log(2.0 * math.pi), jnp.float32)
    for d in range(D_FEAT):
        mu = mu_v[pl.ds(d * K, K)]
        ls = ls_v[pl.ds(d * K, K)]
        iv = jnp.exp(-2.0 * ls)
        c = c - (0.5 * mu * mu) * iv - ls
        mu_v[pl.ds(d * K, K)] = mu * iv
        ls_v[pl.ds(d * K, K)] = -0.5 * iv

    def token_body(t, _):
        x_lo = x_v[pl.ds(t * D_FEAT, 16)]
        x_hi = x_v[pl.ds(t * D_FEAT + 16, 16)]
        acc = c
        for d in range(16):
            xs = x_lo[d]
            acc = acc + mu_v[pl.ds(d * K, K)] * xs + ls_v[pl.ds(d * K, K)] * (xs * xs)
        for d in range(16):
            xs = x_hi[d]
            acc = (acc + mu_v[pl.ds((16 + d) * K, K)] * xs
                   + ls_v[pl.ds((16 + d) * K, K)] * (xs * xs))
        out_v[pl.ds(t * K, K)] = acc
        return 0

    lax.fori_loop(0, CHUNK, token_body, 0)
    pltpu.sync_copy(out_v, out_hbm.at[pl.ds(base * K, CHUNK * K)])


@jax.jit
def _emission_log_probs(X, means, log_scales):
    mesh = plsc.VectorSubcoreMesh(core_axis_name="c", subcore_axis_name="s")
    run = pl.kernel(
        _sc_body,
        out_type=jax.ShapeDtypeStruct((T * K,), jnp.float32),
        mesh=mesh,
        scratch_types=[
            pltpu.VMEM((CHUNK * D_FEAT,), jnp.float32),
            pltpu.VMEM((CHUNK * K,), jnp.float32),
            pltpu.VMEM((D_FEAT * K,), jnp.float32),
            pltpu.VMEM((D_FEAT * K,), jnp.float32),
        ],
    )
    out = run(X.reshape(-1), means.T.reshape(-1), log_scales.T.reshape(-1))
    return out.reshape(T, K)


def kernel(X, cu_seqlens, means, log_scales, pi_logits, A_logits, D_logits):
    return _emission_log_probs(
        X.astype(jnp.float32),
        means.astype(jnp.float32),
        log_scales.astype(jnp.float32),
    )


# speedup vs baseline: 1.5725x; 1.5725x over previous
import math

import jax
import jax.numpy as jnp
from jax import lax
from jax.experimental import pallas as pl
from jax.experimental.pallas import tpu as pltpu
from jax.experimental.pallas import tpu_sc as plsc

T = 32768
D_FEAT = 32
K = 16
NW = 32
CHUNK = T // NW
DB = 8


def _sc_body(x_hbm, mu_t_hbm, ls_t_hbm, out_hbm, x_v, out_v, s_v, m_v):
    wid = lax.axis_index("s") * 2 + lax.axis_index("c")
    base = wid * CHUNK

    pltpu.sync_copy(x_hbm.at[pl.ds(base * D_FEAT, CHUNK * D_FEAT)], x_v)
    pltpu.sync_copy(mu_t_hbm, m_v)
    pltpu.sync_copy(ls_t_hbm, s_v)

    c = jnp.full((K,), -0.5 * D_FEAT * math.log(2.0 * math.pi), jnp.float32)
    for d in range(D_FEAT):
        mu = m_v[pl.ds(d * K, K)]
        ls = s_v[pl.ds(d * K, K)]
        s = jnp.exp(-ls)
        c = c - ls
        s_v[pl.ds(d * K, K)] = s
        m_v[pl.ds(d * K, K)] = mu * s

    num_blocks = D_FEAT // DB
    for db in range(num_blocks):
        s_w = [s_v[pl.ds((db * DB + j) * K, K)] for j in range(DB)]
        m_w = [m_v[pl.ds((db * DB + j) * K, K)] for j in range(DB)]
        half = (db * DB // 16) * 16
        off = db * DB - half

        @plsc.parallel_loop(0, CHUNK, 1, unroll=2)
        def _token_body(t, _db=db, _s=s_w, _m=m_w, _half=half, _off=off):
            xv = x_v[pl.ds(t * D_FEAT + _half, 16)]
            if _db == 0:
                acc = jnp.zeros((K,), jnp.float32)
            else:
                acc = out_v[pl.ds(t * K, K)]
            for j in range(DB):
                z = xv[_off + j] * _s[j] - _m[j]
                acc = acc + z * z
            if _db == num_blocks - 1:
                acc = c - 0.5 * acc
            out_v[pl.ds(t * K, K)] = acc

    pltpu.sync_copy(out_v, out_hbm.at[pl.ds(base * K, CHUNK * K)])


@jax.jit
def _emission_log_probs(X, means, log_scales):
    mesh = plsc.VectorSubcoreMesh(core_axis_name="c", subcore_axis_name="s")
    run = pl.kernel(
        _sc_body,
        out_type=jax.ShapeDtypeStruct((T * K,), jnp.float32),
        mesh=mesh,
        scratch_types=[
            pltpu.VMEM((CHUNK * D_FEAT,), jnp.float32),
            pltpu.VMEM((CHUNK * K,), jnp.float32),
            pltpu.VMEM((D_FEAT * K,), jnp.float32),
            pltpu.VMEM((D_FEAT * K,), jnp.float32),
        ],
    )
    out = run(X.reshape(-1), means.T.reshape(-1), log_scales.T.reshape(-1))
    return out.reshape(T, K)


def kernel(X, cu_seqlens, means, log_scales, pi_logits, A_logits, D_logits):
    return _emission_log_probs(
        X.astype(jnp.float32),
        means.astype(jnp.float32),
        log_scales.astype(jnp.float32),
    )
